# baseline (speedup 1.0000x reference)
import jax
import jax.numpy as jnp
from jax import lax
from jax.experimental import pallas as pl
from jax.experimental.pallas import tpu as pltpu
from jax.experimental.pallas import tpu_sc as plsc

D_MODEL = 768
BATCH = 4
SEQ_LEN = 2048

NUM_CORES = 2
NUM_SUBCORES = 16
NUM_WORKERS = NUM_CORES * NUM_SUBCORES
POS_PER_W = SEQ_LEN // NUM_WORKERS
LANES = 16

HALF = POS_PER_W // 2
NCHUNK = BATCH * 2


def _sc_embed_body(src_hbm, tok_hbm, pos_hbm, out_hbm, idx_v, pos_v, tok_v,
                   gs0, gs1, ws0, ws1):
    gsem = (gs0, gs1)
    wsem = (ws0, ws1)
    cid = lax.axis_index("c")
    sid = lax.axis_index("s")
    wid = sid * NUM_CORES + cid
    p0 = wid * POS_PER_W

    pltpu.sync_copy(pos_hbm.at[pl.ds(p0, POS_PER_W)], pos_v)
    for c in range(NCHUNK):
        b, h = divmod(c, 2)
        pltpu.sync_copy(src_hbm.at[b, pl.ds(p0 + h * HALF, HALF)], idx_v.at[c])

    wb = {}
    for b in range(BATCH):
        g = {}
        for h in range(2):
            if b > 0:
                wb[(b - 1, h)].wait()
            g[h] = pltpu.async_copy(tok_hbm.at[idx_v.at[2 * b + h]],
                                    tok_v.at[pl.ds(h * HALF, HALF)], gsem[h])
        for h in range(2):
            g[h].wait()

            def _row_add(r, carry):
                for j in range(D_MODEL // LANES):
                    sl = pl.ds(j * LANES, LANES)
                    plsc.addupdate(tok_v.at[h * HALF + r, sl], pos_v[h * HALF + r, sl])
                return carry

            lax.fori_loop(0, HALF, _row_add, 0)
            wb[(b, h)] = pltpu.make_async_copy(
                tok_v.at[pl.ds(h * HALF, HALF)],
                out_hbm.at[b, pl.ds(p0 + h * HALF, HALF)], wsem[h])
            wb[(b, h)].start()

    for h in range(2):
        wb[(BATCH - 1, h)].wait()


@jax.jit
def _sc_embed(src, token_table, pos_table):
    mesh = plsc.VectorSubcoreMesh(
        core_axis_name="c",
        subcore_axis_name="s",
        num_cores=NUM_CORES,
        num_subcores=NUM_SUBCORES,
    )
    f = pl.kernel(
        _sc_embed_body,
        out_type=jax.ShapeDtypeStruct((BATCH, SEQ_LEN, D_MODEL), jnp.float32),
        mesh=mesh,
        scratch_types=[
            pltpu.VMEM((NCHUNK, HALF), jnp.int32),
            pltpu.VMEM((POS_PER_W, D_MODEL), jnp.float32),
            pltpu.VMEM((POS_PER_W, D_MODEL), jnp.float32),
        ] + [pltpu.SemaphoreType.DMA] * 4,
    )
    return f(src, token_table, pos_table)


def kernel(src, token_table, pos_table):
    return _sc_embed(src.astype(jnp.int32), token_table, pos_table)
